# baseline (speedup 1.0000x reference)
import functools

import jax
import jax.numpy as jnp
from jax import lax
from jax.experimental import pallas as pl
from jax.experimental.pallas import tpu as pltpu
from jax.experimental.pallas import tpu_sc as plsc

C = 10
N = 16384
D = 64
L = 16
NW = 32
RPW = N // NW
NG = RPW // L
CD = C * D
DP = D + 1
CDP = C * DP + 6
NCH = D // L


def _sqrt16(a):
    i = lax.bitcast_convert_type(a, jnp.int32)
    z = lax.bitcast_convert_type(jnp.int32(0x5F3759DF) - (i >> 1), jnp.float32)
    for _ in range(3):
        z = z * (1.5 - (0.5 * a) * z * z)
    return a * z


def _tc_body(x_ref, tgt_ref, cent_ref, q_ref, prt_ref, mn_ref):
    x = x_ref[...]
    cent = cent_ref[...]
    q = q_ref[...]
    tgt = tgt_ref[...]
    onehot = (tgt[:, None] == lax.broadcasted_iota(jnp.int32, (1, C), 1)
              ).astype(jnp.float32)
    qrow = lax.dot_general(onehot, q, (((1,), (0,)), ((), ())),
                           preferred_element_type=jnp.float32)
    pr = x * qrow
    eye = (lax.broadcasted_iota(jnp.int32, (D, D), 0)
           == lax.broadcasted_iota(jnp.int32, (D, D), 1)).astype(jnp.float32)
    prt = lax.dot_general(eye, pr, (((1,), (1,)), ((), ())),
                          preferred_element_type=jnp.float32)
    g = lax.dot_general(cent, pr, (((1,), (1,)), ((), ())),
                        preferred_element_type=jnp.float32)
    cs2 = jnp.sum(cent * cent, axis=1, keepdims=True)
    r2p = jnp.sum(pr * pr, axis=1)[None, :]
    prt_ref[...] = prt[None]
    mn_ref[...] = jnp.maximum(cs2 - 2.0 * g + r2p, 0.0)[None]


def _pass1_body(
    prt_hbm, tgt_hbm, mn_hbm,
    partA_hbm, partB_hbm,
    prt_v, tgt_v, mn_v, acc_v, accvl_v,
):
    wid = lax.axis_index("s") * 2 + lax.axis_index("c")
    base = wid * RPW

    pltpu.sync_copy(prt_hbm.at[pl.ds(wid * D * RPW, D * RPW)], prt_v)
    pltpu.sync_copy(tgt_hbm.at[pl.ds(base, RPW)], tgt_v)
    pltpu.sync_copy(mn_hbm.at[pl.ds(wid * C * RPW, C * RPW)], mn_v)

    zv = jnp.zeros((L,), jnp.float32)
    for qq in range(CDP // L):
        acc_v[pl.ds(qq * L, L)] = zv
    accvl_v[...] = zv

    lane_iota = lax.iota(jnp.int32, L)

    def grp_body(g, carry):
        rb = g * L
        tvec = tgt_v[pl.ds(rb, L)]
        t65 = tvec * DP
        nsidx = tvec * RPW + (rb + lane_iota)
        ns = plsc.load_gather(mn_v, [nsidx])
        nrm = _sqrt16(ns)
        plsc.addupdate_scatter(accvl_v, [tvec], nrm)
        for f in range(D):
            v = prt_v[pl.ds(f * RPW + rb, L)]
            plsc.addupdate_scatter(acc_v, [t65 + f], v)
        return carry

    lax.fori_loop(0, NG, grp_body, 0)

    pltpu.sync_copy(acc_v, partA_hbm.at[pl.ds(wid * CDP, CDP)])
    pltpu.sync_copy(accvl_v, partB_hbm.at[pl.ds(wid * L, L)])


def _pass2_body(
    cent_hbm, countp_hbm, distp_hbm, wp_hbm, partA_hbm, partB_hbm,
    out_hbm,
    cent_v, countp_v, invc_v, distp_v, wp_v,
    allA_v, allB_v, accp_v, acc_v, tot_v, out_v,
):
    wid = lax.axis_index("s") * 2 + lax.axis_index("c")

    @pl.when(wid == 0)
    def _():
        pltpu.sync_copy(cent_hbm, cent_v)
        pltpu.sync_copy(countp_hbm, countp_v)
        pltpu.sync_copy(distp_hbm, distp_v)
        pltpu.sync_copy(wp_hbm, wp_v)
        pltpu.sync_copy(partA_hbm, allA_v)
        pltpu.sync_copy(partB_hbm, allB_v)

        invc_v[...] = 1.0 / countp_v[...]
        lane_iota = lax.iota(jnp.int32, L)

        zv = jnp.zeros((L,), jnp.float32)
        for qq in range(CDP // L):
            accp_v[pl.ds(qq * L, L)] = zv

        def comb_body(w, carry):
            wb = w * CDP
            for qq in range(CDP // L):
                plsc.addupdate(accp_v.at[pl.ds(qq * L, L)],
                               allA_v[pl.ds(wb + qq * L, L)])
            return carry

        lax.fori_loop(0, NW, comb_body, 0)

        for c in range(C):
            for k in range(NCH):
                idx = c * DP + k * L + lane_iota
                acc_v[pl.ds((c * NCH + k) * L, L)] = (
                    cent_v[pl.ds((c * NCH + k) * L, L)]
                    + plsc.load_gather(accp_v, [idx]))

        sabs = jnp.zeros((L,), jnp.float32)
        for qq in range(CD // L):
            sabs = sabs + jnp.abs(acc_v[pl.ds(qq * L, L)])
        sabs_s = jnp.sum(sabs)

        def svec_body(w, carry):
            plsc.addupdate(distp_v.at[:], allB_v[pl.ds(w * L, L)])
            return carry

        lax.fori_loop(0, NW, svec_body, 0)
        s_vec = _sqrt16(distp_v[...]) * invc_v[...]

        tot = jnp.zeros((L,), jnp.float32)
        for i in range(C):
            ib = i * D
            ci = [acc_v[pl.ds(ib + k * L, L)] for k in range(NCH)]
            d2row = jnp.zeros((L,), jnp.float32)
            for j in range(C):
                if j == i:
                    continue
                s2 = jnp.zeros((L,), jnp.float32)
                for k in range(NCH):
                    dv = ci[k] - acc_v[pl.ds(j * D + k * L, L)]
                    s2 = s2 + dv * dv
                d2row = jnp.where(lane_iota == j, jnp.sum(s2), d2row)
            mask = (lane_iota < C) & (lane_iota != i)
            mrow = _sqrt16(jnp.where(mask, d2row, 1.0))
            numer = wp_v[pl.ds(i * L, L)] * (s_vec[i] + s_vec)
            term = jnp.where(mask, numer / mrow, 0.0)
            tot = tot + term
        tot_v[...] = tot

        total_s = jnp.sum(tot_v[...])
        total_vec = jnp.zeros((L,), jnp.float32) + total_s
        sabs_vec = jnp.zeros((L,), jnp.float32) + sabs_s
        loss_vec = total_vec / float(C) * float(C - 1) + sabs_vec / 1000000.0
        out_v[...] = loss_vec
        pltpu.sync_copy(out_v, out_hbm)


@jax.jit
def _db_loss(pred2d, tgt, cent2d, cent, q2d, countp, distp, wp):
    prt, mn = pl.pallas_call(
        _tc_body,
        grid=(NW,),
        in_specs=[
            pl.BlockSpec((RPW, D), lambda i: (i, 0)),
            pl.BlockSpec((RPW,), lambda i: (i,)),
            pl.BlockSpec((C, D), lambda i: (0, 0)),
            pl.BlockSpec((C, 1), lambda i: (0, 0)),
        ],
        out_specs=[
            pl.BlockSpec((1, D, RPW), lambda i: (i, 0, 0)),
            pl.BlockSpec((1, C, RPW), lambda i: (i, 0, 0)),
        ],
        out_shape=[
            jax.ShapeDtypeStruct((NW, D, RPW), jnp.float32),
            jax.ShapeDtypeStruct((NW, C, RPW), jnp.float32),
        ],
    )(pred2d, tgt, cent2d, q2d)

    mesh = plsc.VectorSubcoreMesh(core_axis_name="c", subcore_axis_name="s")
    params = pltpu.CompilerParams(needs_layout_passes=False)

    pass1 = functools.partial(
        pl.kernel,
        out_type=[
            jax.ShapeDtypeStruct((NW * CDP,), jnp.float32),
            jax.ShapeDtypeStruct((NW * L,), jnp.float32),
        ],
        mesh=mesh,
        compiler_params=params,
        scratch_types=[
            pltpu.VMEM((D * RPW,), jnp.float32),
            pltpu.VMEM((RPW,), jnp.int32),
            pltpu.VMEM((C * RPW,), jnp.float32),
            pltpu.VMEM((CDP,), jnp.float32),
            pltpu.VMEM((L,), jnp.float32),
        ],
    )(_pass1_body)
    partA, partB = pass1(prt.reshape(NW * D * RPW), tgt, mn.reshape(NW * C * RPW))

    pass2 = functools.partial(
        pl.kernel,
        out_type=jax.ShapeDtypeStruct((L,), jnp.float32),
        mesh=mesh,
        compiler_params=params,
        scratch_types=[
            pltpu.VMEM((CD,), jnp.float32),
            pltpu.VMEM((L,), jnp.float32),
            pltpu.VMEM((L,), jnp.float32),
            pltpu.VMEM((L,), jnp.float32),
            pltpu.VMEM((C * L,), jnp.float32),
            pltpu.VMEM((NW * CDP,), jnp.float32),
            pltpu.VMEM((NW * L,), jnp.float32),
            pltpu.VMEM((CDP,), jnp.float32),
            pltpu.VMEM((CD,), jnp.float32),
            pltpu.VMEM((L,), jnp.float32),
            pltpu.VMEM((L,), jnp.float32),
        ],
    )(_pass2_body)
    return pass2(cent, countp, distp, wp, partA, partB)


def kernel(predicted, centroids, count, distances, class_weights_matrix, target, epoch):
    countp = jnp.concatenate([count[:, 0], jnp.ones((L - C,), jnp.float32)])
    distp = jnp.concatenate([distances[:, 0], jnp.zeros((L - C,), jnp.float32)])
    wp = jnp.pad(class_weights_matrix, ((0, 0), (0, L - C))).reshape(C * L)
    q2d = 1.0 / count
    out = _db_loss(predicted, target.astype(jnp.int32),
                   centroids, centroids.reshape(CD), q2d, countp, distp, wp)
    return out[:1]


# speedup vs baseline: 1.6560x; 1.6560x over previous
import functools

import jax
import jax.numpy as jnp
from jax import lax
from jax.experimental import pallas as pl
from jax.experimental.pallas import tpu as pltpu
from jax.experimental.pallas import tpu_sc as plsc

C = 10
N = 16384
D = 64
L = 16
NW = 32
RPW = N // NW
NCH = D // L
NG = RPW // L
CD = C * D


def _sqrt16(a):
    i = lax.bitcast_convert_type(a, jnp.int32)
    z = lax.bitcast_convert_type(jnp.int32(0x5F3759DF) - (i >> 1), jnp.float32)
    for _ in range(3):
        z = z * (1.5 - (0.5 * a) * z * z)
    return a * z


def _pass1_body(
    pred_hbm, tgt_hbm, cent_hbm, countp_hbm,
    partA_hbm, partB_hbm,
    pred_v, tgt_v, cent_v, countp_v, invc_v,
    acc_v, accvl_v, accvec_v,
):
    wid = lax.axis_index("s") * 2 + lax.axis_index("c")
    base = wid * RPW

    pltpu.sync_copy(pred_hbm.at[pl.ds(base * D, RPW * D)], pred_v)
    pltpu.sync_copy(tgt_hbm.at[pl.ds(base, RPW)], tgt_v)
    pltpu.sync_copy(cent_hbm, cent_v)
    pltpu.sync_copy(countp_hbm, countp_v)

    invc_v[...] = 1.0 / countp_v[...]

    zv = jnp.zeros((L,), jnp.float32)
    for q in range(CD // L):
        acc_v[pl.ds(q * L, L)] = zv
    for c in range(C):
        accvl_v[pl.ds(c * L, L)] = zv

    lane_iota = lax.iota(jnp.int32, L)
    invc_all = invc_v[...]

    def grp_body(g, carry):
        tvec = tgt_v[pl.ds(g * L, L)]
        invvec = jnp.zeros((L,), jnp.float32)
        for c in range(C):
            invvec = jnp.where(tvec == c, invc_all[c], invvec)
        svec = jnp.zeros((L,), jnp.float32)
        for lane in range(L):
            cls = tvec[lane]
            inv = invvec[lane]
            rb = (g * L + lane) * D
            cb = cls * D
            s2 = jnp.zeros((L,), jnp.float32)
            for k in range(NCH):
                x = pred_v[pl.ds(rb + k * L, L)]
                pr = x * inv
                diff = cent_v[pl.ds(cb + k * L, L)] - pr
                s2 = s2 + diff * diff
                plsc.addupdate(acc_v.at[pl.ds(cb + k * L, L)], pr)
            svec = jnp.where(lane_iota == lane, jnp.sum(s2), svec)
        nrm = _sqrt16(svec)
        for c in range(C):
            contrib = jnp.where(tvec == c, nrm, 0.0)
            plsc.addupdate(accvl_v.at[pl.ds(c * L, L)], contrib)
        return carry

    lax.fori_loop(0, NG, grp_body, 0)

    avec = jnp.zeros((L,), jnp.float32)
    for c in range(C):
        avec = jnp.where(lane_iota == c, jnp.sum(accvl_v[pl.ds(c * L, L)]), avec)
    accvec_v[...] = avec

    pltpu.sync_copy(acc_v, partA_hbm.at[pl.ds(wid * CD, CD)])
    pltpu.sync_copy(accvec_v, partB_hbm.at[pl.ds(wid * L, L)])


def _tc_tail_body(partA_ref, partB_ref, cent_ref, count_ref, dist_ref, w_ref,
                  out_ref):
    c2 = cent_ref[...] + jnp.sum(partA_ref[...], axis=0)
    nrmsum = jnp.sum(partB_ref[...], axis=0)[:C]
    s = jnp.sqrt(dist_ref[...][:, 0] + nrmsum) / count_ref[...][:, 0]
    gram = lax.dot_general(c2, c2, (((1,), (1,)), ((), ())),
                           preferred_element_type=jnp.float32)
    cs = jnp.sum(c2 * c2, axis=1)
    d2 = cs[:, None] + cs[None, :] - 2.0 * gram
    eye = (lax.broadcasted_iota(jnp.int32, (C, C), 0)
           == lax.broadcasted_iota(jnp.int32, (C, C), 1))
    m = jnp.sqrt(jnp.where(eye, 1.0, jnp.maximum(d2, 0.0)))
    pair = w_ref[...] * (s[:, None] + s[None, :]) / m
    total = jnp.sum(jnp.where(eye, 0.0, pair))
    loss = total / C * (C - 1) + jnp.sum(jnp.abs(c2)) / 1000000.0
    out_ref[...] = jnp.full((1, 1), 0.0) + loss


@jax.jit
def _db_loss(pred, tgt, cent2d, cent, countp, count2d, dist2d, w2d):
    mesh = plsc.VectorSubcoreMesh(core_axis_name="c", subcore_axis_name="s")
    params = pltpu.CompilerParams(needs_layout_passes=False)

    pass1 = functools.partial(
        pl.kernel,
        out_type=[
            jax.ShapeDtypeStruct((NW * CD,), jnp.float32),
            jax.ShapeDtypeStruct((NW * L,), jnp.float32),
        ],
        mesh=mesh,
        compiler_params=params,
        scratch_types=[
            pltpu.VMEM((RPW * D,), jnp.float32),
            pltpu.VMEM((RPW,), jnp.int32),
            pltpu.VMEM((CD,), jnp.float32),
            pltpu.VMEM((L,), jnp.float32),
            pltpu.VMEM((L,), jnp.float32),
            pltpu.VMEM((CD,), jnp.float32),
            pltpu.VMEM((C * L,), jnp.float32),
            pltpu.VMEM((L,), jnp.float32),
        ],
    )(_pass1_body)
    partA, partB = pass1(pred, tgt, cent, countp)

    out = pl.pallas_call(
        _tc_tail_body,
        in_specs=[
            pl.BlockSpec((NW, C, D), lambda: (0, 0, 0)),
            pl.BlockSpec((NW, L), lambda: (0, 0)),
            pl.BlockSpec((C, D), lambda: (0, 0)),
            pl.BlockSpec((C, 1), lambda: (0, 0)),
            pl.BlockSpec((C, 1), lambda: (0, 0)),
            pl.BlockSpec((C, C), lambda: (0, 0)),
        ],
        out_specs=pl.BlockSpec((1, 1), lambda: (0, 0)),
        out_shape=jax.ShapeDtypeStruct((1, 1), jnp.float32),
    )(partA.reshape(NW, C, D), partB.reshape(NW, L), cent2d,
      count2d, dist2d, w2d)
    return out


def kernel(predicted, centroids, count, distances, class_weights_matrix, target, epoch):
    countp = jnp.concatenate([count[:, 0], jnp.ones((L - C,), jnp.float32)])
    out = _db_loss(predicted.reshape(N * D), target.astype(jnp.int32),
                   centroids, centroids.reshape(CD), countp,
                   count, distances, class_weights_matrix)
    return out.reshape(1)
